# baseline (speedup 1.0000x reference)
import functools

import jax
import jax.numpy as jnp
from jax import lax
from jax.experimental import pallas as pl
from jax.experimental.pallas import tpu as pltpu
from jax.experimental.pallas import tpu_sc as plsc

B = 16384
D_NUM = 13
N_CAT = 26
D_TOKEN = 32
N_TOK = 1 + D_NUM + N_CAT
ROW = N_TOK * D_TOKEN

NW = 32
RPT = B // NW
CB = 32
NCH = RPT // CB
NP = NCH // 2
G = 104
NG = CB * N_CAT // G

_mesh = plsc.VectorSubcoreMesh(core_axis_name="c", subcore_axis_name="s")


@functools.partial(
    pl.kernel,
    mesh=_mesh,
    compiler_params=pltpu.CompilerParams(use_tc_tiling_on_sc=False),
    out_type=jax.ShapeDtypeStruct((B * ROW,), jnp.float32),
    scratch_types=[
        pltpu.VMEM((CB * N_CAT,), jnp.int32),
        pltpu.VMEM((CB * N_CAT,), jnp.int32),
        pltpu.VMEM((CB, 16), jnp.float32),
        pltpu.VMEM((1 + D_NUM, D_TOKEN), jnp.float32),
        pltpu.VMEM((D_NUM + N_CAT, D_TOKEN), jnp.float32),
        pltpu.VMEM((CB * N_CAT, D_TOKEN), jnp.float32),
        pltpu.VMEM((CB * N_CAT, D_TOKEN), jnp.float32),
        pltpu.VMEM((CB * ROW,), jnp.float32),
        pltpu.SemaphoreType.DMA,
        pltpu.SemaphoreType.DMA,
        pltpu.SemaphoreType.DMA,
    ],
)
def _tokenizer_sc(table, idxh, xnh, wh, bh, outh,
                  idx0, idx1, xn_v, w_v, b_v, cat0, cat1, out_v,
                  semg0, semg1, semo):
    wid = lax.axis_index("s") * 2 + lax.axis_index("c")
    base = wid * RPT
    pltpu.sync_copy(wh, w_v)
    pltpu.sync_copy(bh, b_v)

    def prefetch(ch, idx_v, cat_v, semg):
        i0 = (base + ch * CB) * N_CAT
        pltpu.sync_copy(idxh.at[pl.ds(i0, CB * N_CAT)], idx_v)
        for k in range(NG):
            pltpu.async_copy(table.at[idx_v.at[pl.ds(k * G, G)]],
                             cat_v.at[pl.ds(k * G, G)], semg)

    def compute(ch, idx_v, cat_v, semg):
        b0 = base + ch * CB
        pltpu.sync_copy(xnh.at[pl.ds(b0, CB)], xn_v)

        @pl.when(ch >= 1)
        def _wait_prev_write():
            bp = (b0 - CB) * ROW
            pltpu.make_async_copy(out_v, outh.at[pl.ds(bp, CB * ROW)],
                                  semo).wait()

        w0l = w_v[0, pl.ds(0, 16)]
        w0h = w_v[0, pl.ds(16, 16)]

        @plsc.parallel_loop(0, CB)
        def _numeric(b):
            o = b * ROW
            out_v[pl.ds(o, 16)] = w0l
            out_v[pl.ds(o + 16, 16)] = w0h
            xr = xn_v[b, pl.ds(0, 16)]
            for j in range(D_NUM):
                s = xr[j]
                r = o + (1 + j) * D_TOKEN
                out_v[pl.ds(r, 16)] = w_v[1 + j, pl.ds(0, 16)] * s + b_v[j, pl.ds(0, 16)]
                out_v[pl.ds(r + 16, 16)] = (
                    w_v[1 + j, pl.ds(16, 16)] * s + b_v[j, pl.ds(16, 16)])

        for h in range(2):
            for k in range(h * NG // 2, (h + 1) * NG // 2):
                pltpu.make_async_copy(table.at[idx_v.at[pl.ds(k * G, G)]],
                                      cat_v.at[pl.ds(k * G, G)], semg).wait()

            @plsc.parallel_loop(h * CB // 2, (h + 1) * CB // 2)
            def _biasadd(b):
                r0 = b * N_CAT
                o = b * ROW + (1 + D_NUM) * D_TOKEN
                for c in range(N_CAT):
                    r = o + c * D_TOKEN
                    out_v[pl.ds(r, 16)] = (
                        cat_v[r0 + c, pl.ds(0, 16)] + b_v[D_NUM + c, pl.ds(0, 16)])
                    out_v[pl.ds(r + 16, 16)] = (
                        cat_v[r0 + c, pl.ds(16, 16)] + b_v[D_NUM + c, pl.ds(16, 16)])

        pltpu.async_copy(out_v, outh.at[pl.ds(b0 * ROW, CB * ROW)], semo)

    prefetch(0, idx0, cat0, semg0)

    def pair_body(i, carry):
        cha = 2 * i
        prefetch(cha + 1, idx1, cat1, semg1)
        compute(cha, idx0, cat0, semg0)

        @pl.when(i < NP - 1)
        def _prefetch_next_pair():
            prefetch(cha + 2, idx0, cat0, semg0)

        compute(cha + 1, idx1, cat1, semg1)
        return carry

    lax.fori_loop(0, NP, pair_body, 0)

    bl = (base + (NCH - 1) * CB) * ROW
    pltpu.make_async_copy(out_v, outh.at[pl.ds(bl, CB * ROW)], semo).wait()


def kernel(x_num, x_cat, weight, bias, emb_table, category_offsets):
    idx = x_cat.astype(jnp.int32) + category_offsets[None, :].astype(jnp.int32)
    xn = jnp.pad(x_num, ((0, 0), (0, 16 - D_NUM)))
    out = _tokenizer_sc(emb_table, idx.reshape(-1), xn, weight, bias)
    return out.reshape(B, N_TOK, D_TOKEN)
